# Initial kernel scaffold; baseline (speedup 1.0000x reference)
import jax
import jax.numpy as jnp
from jax.experimental import pallas as pl


def kernel(hidden_states, gate_up_proj, down_proj, expert_indices, expert_weights):
    raise NotImplementedError("write your pallas kernel here")


# speedup vs baseline: 1.7593x; 1.7593x over previous
import functools

import jax
import jax.numpy as jnp
from jax.experimental import pallas as pl
from jax.experimental.pallas import tpu as pltpu

NUM_EXPERTS = 8
TOP_K = 2
BT = 512


def _moe_block(idx_ref, ew_ref, x_ref, gup_ref, dp_ref, o_ref):
    e = pl.program_id(1)
    x = x_ref[...]
    gu = jnp.dot(x, gup_ref[0], preferred_element_type=jnp.float32)
    half = gu.shape[1] // 2
    g = gu[:, :half]
    u = gu[:, half:]
    act = g * jax.nn.sigmoid(g) * u
    eo = jnp.dot(act, dp_ref[0], preferred_element_type=jnp.float32)
    w = jnp.sum(jnp.where(idx_ref[...] == e, ew_ref[...], 0.0), axis=1)
    contrib = eo * w[:, None]

    @pl.when(e == 0)
    def _init():
        o_ref[...] = contrib

    @pl.when(e > 0)
    def _acc():
        o_ref[...] += contrib


def kernel(hidden_states, gate_up_proj, down_proj, expert_indices, expert_weights):
    B, S, H = hidden_states.shape
    T = B * S
    E, _, I2 = gate_up_proj.shape
    I = I2 // 2
    flat = hidden_states.reshape(T, H)

    grid = (T // BT, E)
    out = pl.pallas_call(
        _moe_block,
        grid=grid,
        in_specs=[
            pl.BlockSpec((BT, TOP_K), lambda t, e: (t, 0)),
            pl.BlockSpec((BT, TOP_K), lambda t, e: (t, 0)),
            pl.BlockSpec((BT, H), lambda t, e: (t, 0)),
            pl.BlockSpec((1, H, I2), lambda t, e: (e, 0, 0)),
            pl.BlockSpec((1, I, H), lambda t, e: (e, 0, 0)),
        ],
        out_specs=pl.BlockSpec((BT, H), lambda t, e: (t, 0)),
        out_shape=jax.ShapeDtypeStruct((T, H), jnp.float32),
    )(
        expert_indices,
        expert_weights,
        flat,
        gate_up_proj,
        down_proj,
    )
    return out.reshape(B, S, H)


# speedup vs baseline: 1.7646x; 1.0030x over previous
import functools

import jax
import jax.numpy as jnp
from jax.experimental import pallas as pl
from jax.experimental.pallas import tpu as pltpu

NUM_EXPERTS = 8
TOP_K = 2
BT = 512


def _moe_block(idx_ref, ew_ref, x_ref, gup_ref, dp_ref, o_ref):
    e = pl.program_id(1)
    x = x_ref[...].astype(jnp.bfloat16)
    gu = jnp.dot(x, gup_ref[0].astype(jnp.bfloat16),
                 preferred_element_type=jnp.float32)
    half = gu.shape[1] // 2
    g = gu[:, :half]
    u = gu[:, half:]
    act = g * jax.nn.sigmoid(g) * u
    eo = jnp.dot(act.astype(jnp.bfloat16), dp_ref[0].astype(jnp.bfloat16),
                 preferred_element_type=jnp.float32)
    w = jnp.sum(jnp.where(idx_ref[...] == e, ew_ref[...], 0.0), axis=1)
    contrib = eo * w[:, None]

    @pl.when(e == 0)
    def _init():
        o_ref[...] = contrib

    @pl.when(e > 0)
    def _acc():
        o_ref[...] += contrib


def kernel(hidden_states, gate_up_proj, down_proj, expert_indices, expert_weights):
    B, S, H = hidden_states.shape
    T = B * S
    E, _, I2 = gate_up_proj.shape
    I = I2 // 2
    flat = hidden_states.reshape(T, H)

    grid = (T // BT, E)
    out = pl.pallas_call(
        _moe_block,
        grid=grid,
        in_specs=[
            pl.BlockSpec((BT, TOP_K), lambda t, e: (t, 0)),
            pl.BlockSpec((BT, TOP_K), lambda t, e: (t, 0)),
            pl.BlockSpec((BT, H), lambda t, e: (t, 0)),
            pl.BlockSpec((1, H, I2), lambda t, e: (e, 0, 0)),
            pl.BlockSpec((1, I, H), lambda t, e: (e, 0, 0)),
        ],
        out_specs=pl.BlockSpec((BT, H), lambda t, e: (t, 0)),
        out_shape=jax.ShapeDtypeStruct((T, H), jnp.float32),
    )(
        expert_indices,
        expert_weights,
        flat,
        gate_up_proj,
        down_proj,
    )
    return out.reshape(B, S, H)


# speedup vs baseline: 1.9422x; 1.1006x over previous
import jax
import jax.numpy as jnp
from jax.experimental import pallas as pl
from jax.experimental.pallas import tpu as pltpu

NUM_EXPERTS = 8
TOP_K = 2
BT = 512


def _moe_expert(idx_ref, ew_ref, x_ref, gup_ref, dp_ref, o_ref):
    e = pl.program_id(0)
    gup = gup_ref[0].astype(jnp.bfloat16)
    dp = dp_ref[0].astype(jnp.bfloat16)
    T = x_ref.shape[0]
    half = gup.shape[1] // 2
    for t in range(T // BT):
        sl = pl.ds(t * BT, BT)
        x = x_ref[sl, :].astype(jnp.bfloat16)
        gu = jnp.dot(x, gup, preferred_element_type=jnp.float32)
        g = gu[:, :half]
        u = gu[:, half:]
        act = g * jax.nn.sigmoid(g) * u
        eo = jnp.dot(act.astype(jnp.bfloat16), dp,
                     preferred_element_type=jnp.float32)
        w = jnp.sum(jnp.where(idx_ref[sl, :] == e, ew_ref[sl, :], 0.0), axis=1)
        contrib = eo * w[:, None]

        @pl.when(e == 0)
        def _init():
            o_ref[sl, :] = contrib

        @pl.when(e > 0)
        def _acc():
            o_ref[sl, :] += contrib


def kernel(hidden_states, gate_up_proj, down_proj, expert_indices, expert_weights):
    B, S, H = hidden_states.shape
    T = B * S
    E, _, I2 = gate_up_proj.shape
    I = I2 // 2
    flat = hidden_states.reshape(T, H)

    out = pl.pallas_call(
        _moe_expert,
        grid=(E,),
        in_specs=[
            pl.BlockSpec((T, TOP_K), lambda e: (0, 0)),
            pl.BlockSpec((T, TOP_K), lambda e: (0, 0)),
            pl.BlockSpec((T, H), lambda e: (0, 0)),
            pl.BlockSpec((1, H, I2), lambda e: (e, 0, 0)),
            pl.BlockSpec((1, I, H), lambda e: (e, 0, 0)),
        ],
        out_specs=pl.BlockSpec((T, H), lambda e: (0, 0)),
        out_shape=jax.ShapeDtypeStruct((T, H), jnp.float32),
    )(
        expert_indices,
        expert_weights,
        flat,
        gate_up_proj,
        down_proj,
    )
    return out.reshape(B, S, H)
